# Initial kernel scaffold; baseline (speedup 1.0000x reference)
import functools

import jax
import jax.numpy as jnp
from jax import lax
from jax.experimental import pallas as pl
from jax.experimental.pallas import tpu as pltpu
from jax.experimental.pallas import tpu_sc as plsc

N = 10000
NP = 10240
E = 320000
DF = 128
DH = 16
NCLS = 40

NC = 2
NS = 16
NT = NC * NS
BW = 128
NB = 80
EPW = NB * BW
EP = NT * EPW
STRIPE = NP // NS

_MESH = plsc.VectorSubcoreMesh(core_axis_name="c", subcore_axis_name="s")


def _splat16(v, lane):
    idx = jnp.full((16,), lane, jnp.int32)
    return jnp.take(v, idx, mode="promise_in_bounds")


def _zero_rows(ref, nrows):
    def body(i, _):
        ref[i, :] = jnp.zeros((DH,), jnp.float32)
        return 0
    lax.fori_loop(0, nrows, body, 0)


def _agg_loop(row2d, col2d, norm2d, h_sh, acc_sh, gbufs, sbufs, gsems, ssems):
    pltpu.async_copy(h_sh.at[row2d.at[0]], gbufs[0], gsems[0])

    def body(j2, _):
        for b in (0, 1):
            j = j2 * 2 + b
            pltpu.make_async_copy(h_sh.at[row2d.at[j]], gbufs[b], gsems[b]).wait()
            if b == 0:
                pltpu.async_copy(h_sh.at[row2d.at[j + 1]], gbufs[1], gsems[1])
            else:
                @pl.when(j2 < NB // 2 - 1)
                def _():
                    pltpu.async_copy(h_sh.at[row2d.at[j + 1]], gbufs[0], gsems[0])
            @pl.when(j2 >= 1)
            def _():
                pltpu.make_async_copy(sbufs[b], acc_sh.at[col2d.at[j]],
                                      ssems[b]).wait()
            def scale(k, _2):
                nv = norm2d[j, pl.ds(k * 16, 16)]
                for l in range(16):
                    m = k * 16 + l
                    sbufs[b][m, :] = gbufs[b][m, :] * _splat16(nv, l)
                return 0
            lax.fori_loop(0, BW // 16, scale, 0)
            pltpu.async_copy(sbufs[b], acc_sh.at[col2d.at[j]], ssems[b],
                             add=True)
        return 0

    lax.fori_loop(0, NB // 2, body, 0)
    pltpu.make_async_copy(sbufs[0], acc_sh.at[col2d.at[NB - 2]], ssems[0]).wait()
    pltpu.make_async_copy(sbufs[1], acc_sh.at[col2d.at[NB - 1]], ssems[1]).wait()


@functools.partial(
    pl.kernel,
    out_type=(
        jax.ShapeDtypeStruct((NC, NP, DH), jnp.float32),
        jax.ShapeDtypeStruct((NP,), jnp.float32),
        jax.ShapeDtypeStruct((NT, NB, BW), jnp.float32),
    ),
    mesh=_MESH,
    scratch_types=[
        pltpu.VMEM_SHARED((NP, DH), jnp.float32),
        pltpu.VMEM_SHARED((NP, DH), jnp.float32),
        pltpu.VMEM_SHARED((NS, NP), jnp.float32),
        pltpu.VMEM_SHARED((NP,), jnp.float32),
        pltpu.VMEM((NB, BW), jnp.int32),
        pltpu.VMEM((NB, BW), jnp.int32),
        pltpu.VMEM((NB, BW), jnp.float32),
        pltpu.VMEM((NB, BW), jnp.float32),
        pltpu.VMEM((NB, BW), jnp.int32),
        pltpu.VMEM((NB, BW), jnp.float32),
        pltpu.VMEM((NP,), jnp.float32),
        pltpu.VMEM((NS, STRIPE), jnp.float32),
        pltpu.VMEM((NP,), jnp.float32),
        pltpu.VMEM((BW, DH), jnp.float32),
        pltpu.VMEM((BW, DH), jnp.float32),
        pltpu.VMEM((BW, DH), jnp.float32),
        pltpu.VMEM((BW, DH), jnp.float32),
        pltpu.SemaphoreType.DMA,
        pltpu.SemaphoreType.DMA,
        pltpu.SemaphoreType.DMA,
        pltpu.SemaphoreType.DMA,
    ],
)
def _sc_layer1(row_h, col_h, ew_h, h0_h, p_h, dis_h, norm_h,
               h_sh, acc_sh, deg_sh, dis_sh,
               row2d, col2d, ew2d, norm2d, col_o, ew_o,
               degl, degbuf, disf, gb0, gb1, sb0, sb1,
               gsem0, gsem1, ssem0, ssem1):
    c = lax.axis_index("c")
    s = lax.axis_index("s")
    wid = s * 2 + c
    owid = s * 2 + (1 - c)
    st = s * STRIPE

    pltpu.sync_copy(row_h.at[wid], row2d)
    pltpu.sync_copy(col_h.at[wid], col2d)
    pltpu.sync_copy(ew_h.at[wid], ew2d)
    pltpu.sync_copy(col_h.at[owid], col_o)
    pltpu.sync_copy(ew_h.at[owid], ew_o)
    pltpu.sync_copy(h0_h.at[pl.ds(st, STRIPE)], h_sh.at[pl.ds(st, STRIPE)])
    _zero_rows(gb0, BW)
    for k in range(STRIPE // BW):
        pltpu.sync_copy(gb0, acc_sh.at[pl.ds(st + k * BW, BW)])

    def zdeg(i, _):
        degl[pl.ds(i * 16, 16)] = jnp.zeros((16,), jnp.float32)
        return 0
    lax.fori_loop(0, NP // 16, zdeg, 0)

    def hist(colref, ewref):
        def body(j, _):
            def inner(k, _2):
                idx = colref[j, pl.ds(k * 16, 16)]
                w = ewref[j, pl.ds(k * 16, 16)]
                plsc.addupdate_scatter(degl, [idx], w)
                return 0
            lax.fori_loop(0, BW // 16, inner, 0)
            return 0
        lax.fori_loop(0, NB, body, 0)

    hist(col2d, ew2d)
    hist(col_o, ew_o)
    pltpu.sync_copy(degl, deg_sh.at[s])
    plsc.subcore_barrier()

    pltpu.sync_copy(deg_sh.at[:, pl.ds(st, STRIPE)], degbuf)

    def newton(v, _):
        dv = jnp.full((16,), 1.0, jnp.float32)
        for t in range(NS):
            dv = dv + degbuf[t, pl.ds(v * 16, 16)]
        bits = plsc.bitcast(dv, jnp.int32)
        y = plsc.bitcast(jnp.full((16,), 0x5F3759DF, jnp.int32) - (bits >> 1),
                         jnp.float32)
        half = dv * 0.5
        for _i in range(4):
            y = y * (1.5 - half * y * y)
        degbuf[0, pl.ds(v * 16, 16)] = y
        return 0
    lax.fori_loop(0, STRIPE // 16, newton, 0)
    pltpu.sync_copy(degbuf.at[0], dis_sh.at[pl.ds(st, STRIPE)])

    @pl.when(c == 0)
    def _():
        pltpu.sync_copy(degbuf.at[0], dis_h.at[pl.ds(st, STRIPE)])
    plsc.subcore_barrier()

    pltpu.sync_copy(dis_sh, disf)

    def nrm(j, _):
        def inner(k, _2):
            sl = pl.ds(k * 16, 16)
            dr = plsc.load_gather(disf, [row2d[j, sl]])
            dc = plsc.load_gather(disf, [col2d[j, sl]])
            norm2d[j, sl] = dr * ew2d[j, sl] * dc
            return 0
        lax.fori_loop(0, BW // 16, inner, 0)
        return 0
    lax.fori_loop(0, NB, nrm, 0)
    pltpu.sync_copy(norm2d, norm_h.at[wid])

    _agg_loop(row2d, col2d, norm2d, h_sh, acc_sh,
              (gb0, gb1), (sb0, sb1), (gsem0, gsem1), (ssem0, ssem1))
    plsc.subcore_barrier()
    pltpu.sync_copy(acc_sh.at[pl.ds(st, STRIPE)],
                    p_h.at[c].at[pl.ds(st, STRIPE)])


@functools.partial(
    pl.kernel,
    out_type=(
        jax.ShapeDtypeStruct((NC, NP, DH), jnp.float32),
        jax.ShapeDtypeStruct((NP, DH), jnp.float32),
    ),
    mesh=_MESH,
    scratch_types=[
        pltpu.VMEM_SHARED((NP, DH), jnp.float32),
        pltpu.VMEM_SHARED((NP, DH), jnp.float32),
        pltpu.VMEM((NB, BW), jnp.int32),
        pltpu.VMEM((NB, BW), jnp.int32),
        pltpu.VMEM((NB, BW), jnp.float32),
        pltpu.VMEM((STRIPE, DH), jnp.float32),
        pltpu.VMEM((STRIPE, DH), jnp.float32),
        pltpu.VMEM((STRIPE, DH), jnp.float32),
        pltpu.VMEM((STRIPE,), jnp.float32),
        pltpu.VMEM((BW, DH), jnp.float32),
        pltpu.VMEM((BW, DH), jnp.float32),
        pltpu.VMEM((BW, DH), jnp.float32),
        pltpu.VMEM((BW, DH), jnp.float32),
        pltpu.SemaphoreType.DMA,
        pltpu.SemaphoreType.DMA,
        pltpu.SemaphoreType.DMA,
        pltpu.SemaphoreType.DMA,
    ],
)
def _sc_layer2(row_h, col_h, norm_h, p_h, h0_h, dis_h, q_h, h1_h,
               h_sh, acc_sh,
               row2d, col2d, norm2d, p0b, p1b, h0b, disb,
               gb0, gb1, sb0, sb1, gsem0, gsem1, ssem0, ssem1):
    c = lax.axis_index("c")
    s = lax.axis_index("s")
    wid = s * 2 + c
    st = s * STRIPE

    pltpu.sync_copy(row_h.at[wid], row2d)
    pltpu.sync_copy(col_h.at[wid], col2d)
    pltpu.sync_copy(norm_h.at[wid], norm2d)
    pltpu.sync_copy(p_h.at[0].at[pl.ds(st, STRIPE)], p0b)
    pltpu.sync_copy(p_h.at[1].at[pl.ds(st, STRIPE)], p1b)
    pltpu.sync_copy(h0_h.at[pl.ds(st, STRIPE)], h0b)
    pltpu.sync_copy(dis_h.at[pl.ds(st, STRIPE)], disb)

    def comb(v, _):
        dv = disb[pl.ds(v * 16, 16)]
        d2 = dv * dv
        for l in range(16):
            m = v * 16 + l
            p0b[m, :] = p0b[m, :] + p1b[m, :] + _splat16(d2, l) * h0b[m, :]
        return 0
    lax.fori_loop(0, STRIPE // 16, comb, 0)
    pltpu.sync_copy(p0b, h_sh.at[pl.ds(st, STRIPE)])

    @pl.when(c == 0)
    def _():
        pltpu.sync_copy(p0b, h1_h.at[pl.ds(st, STRIPE)])

    _zero_rows(gb0, BW)
    for k in range(STRIPE // BW):
        pltpu.sync_copy(gb0, acc_sh.at[pl.ds(st + k * BW, BW)])
    plsc.subcore_barrier()

    _agg_loop(row2d, col2d, norm2d, h_sh, acc_sh,
              (gb0, gb1), (sb0, sb1), (gsem0, gsem1), (ssem0, ssem1))
    plsc.subcore_barrier()
    pltpu.sync_copy(acc_sh.at[pl.ds(st, STRIPE)],
                    q_h.at[c].at[pl.ds(st, STRIPE)])


def _mm_body(x_ref, w_ref, o_ref):
    o_ref[...] = jnp.dot(x_ref[...], w_ref[...],
                         preferred_element_type=jnp.float32)


_tc_matmul = pl.pallas_call(
    _mm_body,
    out_shape=jax.ShapeDtypeStruct((NP, DH), jnp.float32),
)


def _final_body(q_ref, h1_ref, dis_ref, w2_ref, o_ref):
    d2 = dis_ref[...] * dis_ref[...]
    h2 = q_ref[0] + q_ref[1] + d2 * h1_ref[...]
    logits = jnp.dot(h2, w2_ref[...], preferred_element_type=jnp.float32)
    m = jnp.max(logits, axis=-1, keepdims=True)
    sh = logits - m
    lse = jnp.log(jnp.sum(jnp.exp(sh), axis=-1, keepdims=True))
    o_ref[...] = sh - lse


_tc_final = pl.pallas_call(
    _final_body,
    out_shape=jax.ShapeDtypeStruct((NP, NCLS), jnp.float32),
)


def kernel(x, edge_index, edge_weight, W1, W2):
    row = edge_index[0].astype(jnp.int32)
    col = edge_index[1].astype(jnp.int32)
    pad = EP - E
    rowp = jnp.concatenate([row, jnp.zeros((pad,), jnp.int32)]).reshape(NT, NB, BW)
    colp = jnp.concatenate([col, jnp.zeros((pad,), jnp.int32)]).reshape(NT, NB, BW)
    ewp = jnp.concatenate(
        [edge_weight.astype(jnp.float32), jnp.zeros((pad,), jnp.float32)]
    ).reshape(NT, NB, BW)
    xp = jnp.pad(x.astype(jnp.float32), ((0, NP - N), (0, 0)))

    h0 = _tc_matmul(xp, W1)
    p, dis, normv = _sc_layer1(rowp, colp, ewp, h0)
    q, h1 = _sc_layer2(rowp, colp, normv, p, h0, dis)
    out = _tc_final(q, h1, dis.reshape(NP, 1), W2)
    return out[:N]


# speedup vs baseline: 27.5423x; 27.5423x over previous
import functools

import jax
import jax.numpy as jnp
from jax import lax
from jax.experimental import pallas as pl
from jax.experimental.pallas import tpu as pltpu
from jax.experimental.pallas import tpu_sc as plsc

N = 10000
NP = 10240
E = 320000
DF = 128
DH = 16
NCLS = 40

NC = 2
NS = 16
NT = NC * NS
BW = 128
NB = 80
EP = NT * NB * BW
STRIPE = NP // NS

_MESH = plsc.VectorSubcoreMesh(core_axis_name="c", subcore_axis_name="s")


def _splat16(v, lane):
    idx = jnp.full((16, 1), lane, jnp.int32)
    dn = lax.GatherDimensionNumbers(
        offset_dims=(), collapsed_slice_dims=(0,), start_index_map=(0,))
    return lax.gather(v, idx, dn, (1,),
                      mode=lax.GatherScatterMode.PROMISE_IN_BOUNDS)


def _zero_rows(ref, nrows):
    def body(i, _):
        ref[i, :] = jnp.zeros((DH,), jnp.float32)
        return 0
    lax.fori_loop(0, nrows, body, 0)


def _agg_loop(row2d, col2d, norm2d, tbl_h, acc_sh, gbuf, sbuf):
    def body(j, _):
        pltpu.sync_copy(tbl_h.at[row2d.at[j]], gbuf)

        def scale(k, _2):
            nv = norm2d[j, pl.ds(k * 16, 16)]
            for l in range(16):
                m = k * 16 + l
                sbuf[m, :] = gbuf[m, :] * _splat16(nv, l)
            return 0
        lax.fori_loop(0, BW // 16, scale, 0)

        pltpu.sync_copy(sbuf, acc_sh.at[col2d.at[j]], add=True)
        return 0

    lax.fori_loop(0, NB, body, 0)


@functools.partial(
    pl.kernel,
    out_type=(
        jax.ShapeDtypeStruct((NC, NP, DH), jnp.float32),
        jax.ShapeDtypeStruct((NP,), jnp.float32),
        jax.ShapeDtypeStruct((NT, NB, BW), jnp.float32),
    ),
    mesh=_MESH,
    compiler_params=pltpu.CompilerParams(needs_layout_passes=False, use_tc_tiling_on_sc=False),
    scratch_types=[
        pltpu.VMEM_SHARED((NP, DH), jnp.float32),
        pltpu.VMEM_SHARED((NP,), jnp.float32),
        pltpu.VMEM_SHARED((NP,), jnp.float32),
        pltpu.VMEM((NB, BW), jnp.int32),
        pltpu.VMEM((NB, BW), jnp.int32),
        pltpu.VMEM((NB, BW), jnp.float32),
        pltpu.VMEM((8, BW), jnp.int32),
        pltpu.VMEM((8, BW), jnp.float32),
        pltpu.VMEM((STRIPE,), jnp.float32),
        pltpu.VMEM((NP,), jnp.float32),
        pltpu.VMEM((BW, DH), jnp.float32),
        pltpu.VMEM((BW, DH), jnp.float32),
    ],
)
def _sc_layer1(row_h, col_h, ew_h, h0_h, p_h, dis_h, norm_h,
               acc_sh, deg_sh, dis_sh,
               row2d, col2d, ew2d, col_w, ew_w,
               degb, disf, gbuf, sbuf):
    c = lax.axis_index("c")
    s = lax.axis_index("s")
    wid = s * 2 + c
    owid = s * 2 + (1 - c)
    st = s * STRIPE

    pltpu.sync_copy(row_h.at[wid], row2d)
    pltpu.sync_copy(col_h.at[wid], col2d)
    pltpu.sync_copy(ew_h.at[wid], ew2d)
    _zero_rows(gbuf, BW)
    for k in range(STRIPE // BW):
        pltpu.sync_copy(gbuf, acc_sh.at[pl.ds(st + k * BW, BW)])
    def ones(i, _):
        degb[pl.ds(i * 16, 16)] = jnp.full((16,), 1.0, jnp.float32)
        return 0
    lax.fori_loop(0, STRIPE // 16, ones, 0)
    pltpu.sync_copy(degb, deg_sh.at[pl.ds(st, STRIPE)])
    plsc.subcore_barrier()

    def hist_mine(j, _):
        pltpu.sync_copy(ew2d.at[j], deg_sh.at[col2d.at[j]], add=True)
        return 0
    lax.fori_loop(0, NB, hist_mine, 0)

    def hist_part(o, _):
        pltpu.sync_copy(col_h.at[owid].at[pl.ds(o * 8, 8)], col_w)
        pltpu.sync_copy(ew_h.at[owid].at[pl.ds(o * 8, 8)], ew_w)
        def inner(i, _2):
            pltpu.sync_copy(ew_w.at[i], deg_sh.at[col_w.at[i]], add=True)
            return 0
        lax.fori_loop(0, 8, inner, 0)
        return 0
    lax.fori_loop(0, NB // 8, hist_part, 0)
    plsc.subcore_barrier()

    pltpu.sync_copy(deg_sh.at[pl.ds(st, STRIPE)], degb)

    def newton(v, _):
        sl = pl.ds(v * 16, 16)
        dv = degb[sl]
        bits = lax.bitcast_convert_type(dv, jnp.int32)
        y = lax.bitcast_convert_type(
            jnp.full((16,), 0x5F3759DF, jnp.int32) - (bits >> 1), jnp.float32)
        half = dv * 0.5
        for _i in range(4):
            y = y * (1.5 - half * y * y)
        degb[sl] = y
        return 0
    lax.fori_loop(0, STRIPE // 16, newton, 0)
    pltpu.sync_copy(degb, dis_sh.at[pl.ds(st, STRIPE)])

    @pl.when(c == 0)
    def _():
        pltpu.sync_copy(degb, dis_h.at[pl.ds(st, STRIPE)])
    plsc.subcore_barrier()

    pltpu.sync_copy(dis_sh, disf)

    def nrm(j, _):
        def inner(k, _2):
            sl = pl.ds(k * 16, 16)
            dr = plsc.load_gather(disf, [row2d[j, sl]])
            dc = plsc.load_gather(disf, [col2d[j, sl]])
            ew2d[j, sl] = dr * ew2d[j, sl] * dc
            return 0
        lax.fori_loop(0, BW // 16, inner, 0)
        return 0
    lax.fori_loop(0, NB, nrm, 0)
    pltpu.sync_copy(ew2d, norm_h.at[wid])

    _agg_loop(row2d, col2d, ew2d, h0_h, acc_sh, gbuf, sbuf)
    plsc.subcore_barrier()
    pltpu.sync_copy(acc_sh.at[pl.ds(st, STRIPE)],
                    p_h.at[c].at[pl.ds(st, STRIPE)])


@functools.partial(
    pl.kernel,
    out_type=jax.ShapeDtypeStruct((NC, NP, DH), jnp.float32),
    mesh=_MESH,
    compiler_params=pltpu.CompilerParams(needs_layout_passes=False, use_tc_tiling_on_sc=False),
    scratch_types=[
        pltpu.VMEM_SHARED((NP, DH), jnp.float32),
        pltpu.VMEM((NB, BW), jnp.int32),
        pltpu.VMEM((NB, BW), jnp.int32),
        pltpu.VMEM((NB, BW), jnp.float32),
        pltpu.VMEM((BW, DH), jnp.float32),
        pltpu.VMEM((BW, DH), jnp.float32),
    ],
)
def _sc_layer2(row_h, col_h, norm_h, h1_h, q_h,
               acc_sh, row2d, col2d, norm2d, gbuf, sbuf):
    c = lax.axis_index("c")
    s = lax.axis_index("s")
    wid = s * 2 + c
    st = s * STRIPE

    pltpu.sync_copy(row_h.at[wid], row2d)
    pltpu.sync_copy(col_h.at[wid], col2d)
    pltpu.sync_copy(norm_h.at[wid], norm2d)
    _zero_rows(gbuf, BW)
    for k in range(STRIPE // BW):
        pltpu.sync_copy(gbuf, acc_sh.at[pl.ds(st + k * BW, BW)])
    plsc.subcore_barrier()

    _agg_loop(row2d, col2d, norm2d, h1_h, acc_sh, gbuf, sbuf)
    plsc.subcore_barrier()
    pltpu.sync_copy(acc_sh.at[pl.ds(st, STRIPE)],
                    q_h.at[c].at[pl.ds(st, STRIPE)])


def _mm_body(x_ref, w_ref, o_ref):
    o_ref[...] = jnp.dot(x_ref[...], w_ref[...],
                         preferred_element_type=jnp.float32)


_tc_matmul = pl.pallas_call(
    _mm_body,
    out_shape=jax.ShapeDtypeStruct((NP, DH), jnp.float32),
)


def _comb_body(p_ref, dis_ref, h0_ref, o_ref):
    d2 = dis_ref[...] * dis_ref[...]
    o_ref[...] = p_ref[0] + p_ref[1] + d2 * h0_ref[...]


_tc_combine = pl.pallas_call(
    _comb_body,
    out_shape=jax.ShapeDtypeStruct((NP, DH), jnp.float32),
)


def _final_body(q_ref, h1_ref, dis_ref, w2_ref, o_ref):
    d2 = dis_ref[...] * dis_ref[...]
    h2 = q_ref[0] + q_ref[1] + d2 * h1_ref[...]
    logits = jnp.dot(h2, w2_ref[...], preferred_element_type=jnp.float32)
    m = jnp.max(logits, axis=-1, keepdims=True)
    sh = logits - m
    lse = jnp.log(jnp.sum(jnp.exp(sh), axis=-1, keepdims=True))
    o_ref[...] = sh - lse


_tc_final = pl.pallas_call(
    _final_body,
    out_shape=jax.ShapeDtypeStruct((NP, NCLS), jnp.float32),
)


def kernel(x, edge_index, edge_weight, W1, W2):
    row = edge_index[0].astype(jnp.int32)
    col = edge_index[1].astype(jnp.int32)
    pad = EP - E
    rowp = jnp.concatenate([row, jnp.zeros((pad,), jnp.int32)]).reshape(NT, NB, BW)
    colp = jnp.concatenate([col, jnp.zeros((pad,), jnp.int32)]).reshape(NT, NB, BW)
    ewp = jnp.concatenate(
        [edge_weight.astype(jnp.float32), jnp.zeros((pad,), jnp.float32)]
    ).reshape(NT, NB, BW)
    xp = jnp.pad(x.astype(jnp.float32), ((0, NP - N), (0, 0)))

    h0 = _tc_matmul(xp, W1)
    p, dis, normv = _sc_layer1(rowp, colp, ewp, h0)
    dis2d = dis.reshape(NP, 1)
    h1 = _tc_combine(p, dis2d, h0)
    q = _sc_layer2(rowp, colp, normv, h1)
    out = _tc_final(q, h1, dis2d, W2)
    return out[:N]


# speedup vs baseline: 30.7479x; 1.1164x over previous
import functools

import jax
import jax.numpy as jnp
from jax import lax
from jax.experimental import pallas as pl
from jax.experimental.pallas import tpu as pltpu
from jax.experimental.pallas import tpu_sc as plsc

N = 10000
NP = 10240
E = 320000
DF = 128
DH = 16
NCLS = 40

NC = 2
NS = 16
NT = NC * NS
BW = 128
NB = 80
EP = NT * NB * BW
STRIPE = NP // NS

_MESH = plsc.VectorSubcoreMesh(core_axis_name="c", subcore_axis_name="s")


def _splat16(v, lane):
    idx = jnp.full((16, 1), lane, jnp.int32)
    dn = lax.GatherDimensionNumbers(
        offset_dims=(), collapsed_slice_dims=(0,), start_index_map=(0,))
    return lax.gather(v, idx, dn, (1,),
                      mode=lax.GatherScatterMode.PROMISE_IN_BOUNDS)


def _zero_rows(ref, nrows):
    def body(i, _):
        ref[i, :] = jnp.zeros((DH,), jnp.float32)
        return 0
    lax.fori_loop(0, nrows, body, 0)


def _agg_loop(row2d, col2d, norm2d, tbl_h, acc_sh,
              gb0, gb1, sb0, sb1, gsem0, gsem1, ssem0, ssem1):
    pltpu.async_copy(tbl_h.at[row2d.at[0]], gb0, gsem0)

    def body(j2, _):
        for b in (0, 1):
            j = j2 * 2 + b
            gb, gs = (gb0, gsem0) if b == 0 else (gb1, gsem1)
            sb, ss = (sb0, ssem0) if b == 0 else (sb1, ssem1)
            ngb, ngs = (gb1, gsem1) if b == 0 else (gb0, gsem0)
            pltpu.make_async_copy(tbl_h.at[row2d.at[j]], gb, gs).wait()
            if b == 0:
                pltpu.async_copy(tbl_h.at[row2d.at[j + 1]], ngb, ngs)
            else:
                @pl.when(j2 < NB // 2 - 1)
                def _():
                    pltpu.async_copy(tbl_h.at[row2d.at[j + 1]], ngb, ngs)
            @pl.when(j2 >= 1)
            def _():
                pltpu.make_async_copy(sb, acc_sh.at[col2d.at[j]], ss).wait()

            def scale(k, _2):
                nv = norm2d[j, pl.ds(k * 16, 16)]
                for l in range(16):
                    m = k * 16 + l
                    sb[m, :] = gb[m, :] * _splat16(nv, l)
                return 0
            lax.fori_loop(0, BW // 16, scale, 0)

            pltpu.async_copy(sb, acc_sh.at[col2d.at[j]], ss, add=True)
        return 0

    lax.fori_loop(0, NB // 2, body, 0)
    pltpu.make_async_copy(sb0, acc_sh.at[col2d.at[NB - 2]], ssem0).wait()
    pltpu.make_async_copy(sb1, acc_sh.at[col2d.at[NB - 1]], ssem1).wait()


@functools.partial(
    pl.kernel,
    out_type=(
        jax.ShapeDtypeStruct((NC, NP, DH), jnp.float32),
        jax.ShapeDtypeStruct((NP,), jnp.float32),
        jax.ShapeDtypeStruct((NT, NB, BW), jnp.float32),
    ),
    mesh=_MESH,
    compiler_params=pltpu.CompilerParams(needs_layout_passes=False, use_tc_tiling_on_sc=False),
    scratch_types=[
        pltpu.VMEM_SHARED((NP, DH), jnp.float32),
        pltpu.VMEM_SHARED((NP,), jnp.float32),
        pltpu.VMEM_SHARED((NP,), jnp.float32),
        pltpu.VMEM((NB, BW), jnp.int32),
        pltpu.VMEM((NB, BW), jnp.int32),
        pltpu.VMEM((NB, BW), jnp.float32),
        pltpu.VMEM((8, BW), jnp.int32),
        pltpu.VMEM((8, BW), jnp.float32),
        pltpu.VMEM((STRIPE,), jnp.float32),
        pltpu.VMEM((NP,), jnp.float32),
        pltpu.VMEM((BW, DH), jnp.float32),
        pltpu.VMEM((BW, DH), jnp.float32),
        pltpu.VMEM((BW, DH), jnp.float32),
        pltpu.VMEM((BW, DH), jnp.float32),
        pltpu.SemaphoreType.DMA,
        pltpu.SemaphoreType.DMA,
        pltpu.SemaphoreType.DMA,
        pltpu.SemaphoreType.DMA,
    ],
)
def _sc_layer1(row_h, col_h, ew_h, h0_h, p_h, dis_h, norm_h,
               acc_sh, deg_sh, dis_sh,
               row2d, col2d, ew2d, col_w, ew_w,
               degb, disf, gb0, gb1, sb0, sb1,
               gsem0, gsem1, ssem0, ssem1):
    c = lax.axis_index("c")
    s = lax.axis_index("s")
    wid = s * 2 + c
    owid = s * 2 + (1 - c)
    st = s * STRIPE

    pltpu.sync_copy(row_h.at[wid], row2d)
    pltpu.sync_copy(col_h.at[wid], col2d)
    pltpu.sync_copy(ew_h.at[wid], ew2d)
    _zero_rows(gb0, BW)
    for k in range(STRIPE // BW):
        pltpu.sync_copy(gb0, acc_sh.at[pl.ds(st + k * BW, BW)])
    def ones(i, _):
        degb[pl.ds(i * 16, 16)] = jnp.full((16,), 1.0, jnp.float32)
        return 0
    lax.fori_loop(0, STRIPE // 16, ones, 0)
    pltpu.sync_copy(degb, deg_sh.at[pl.ds(st, STRIPE)])
    plsc.subcore_barrier()

    def hist_mine(j, _):
        pltpu.sync_copy(ew2d.at[j], deg_sh.at[col2d.at[j]], add=True)
        return 0
    lax.fori_loop(0, NB, hist_mine, 0)

    def hist_part(o, _):
        pltpu.sync_copy(col_h.at[owid].at[pl.ds(o * 8, 8)], col_w)
        pltpu.sync_copy(ew_h.at[owid].at[pl.ds(o * 8, 8)], ew_w)
        def inner(i, _2):
            pltpu.sync_copy(ew_w.at[i], deg_sh.at[col_w.at[i]], add=True)
            return 0
        lax.fori_loop(0, 8, inner, 0)
        return 0
    lax.fori_loop(0, NB // 8, hist_part, 0)
    plsc.subcore_barrier()

    pltpu.sync_copy(deg_sh.at[pl.ds(st, STRIPE)], degb)

    def newton(v, _):
        sl = pl.ds(v * 16, 16)
        dv = degb[sl]
        bits = lax.bitcast_convert_type(dv, jnp.int32)
        y = lax.bitcast_convert_type(
            jnp.full((16,), 0x5F3759DF, jnp.int32) - (bits >> 1), jnp.float32)
        half = dv * 0.5
        for _i in range(4):
            y = y * (1.5 - half * y * y)
        degb[sl] = y
        return 0
    lax.fori_loop(0, STRIPE // 16, newton, 0)
    pltpu.sync_copy(degb, dis_sh.at[pl.ds(st, STRIPE)])

    @pl.when(c == 0)
    def _():
        pltpu.sync_copy(degb, dis_h.at[pl.ds(st, STRIPE)])
    plsc.subcore_barrier()

    pltpu.sync_copy(dis_sh, disf)

    def nrm(j, _):
        def inner(k, _2):
            sl = pl.ds(k * 16, 16)
            dr = plsc.load_gather(disf, [row2d[j, sl]])
            dc = plsc.load_gather(disf, [col2d[j, sl]])
            ew2d[j, sl] = dr * ew2d[j, sl] * dc
            return 0
        lax.fori_loop(0, BW // 16, inner, 0)
        return 0
    lax.fori_loop(0, NB, nrm, 0)
    pltpu.sync_copy(ew2d, norm_h.at[wid])

    _agg_loop(row2d, col2d, ew2d, h0_h, acc_sh,
              gb0, gb1, sb0, sb1, gsem0, gsem1, ssem0, ssem1)
    plsc.subcore_barrier()
    pltpu.sync_copy(acc_sh.at[pl.ds(st, STRIPE)],
                    p_h.at[c].at[pl.ds(st, STRIPE)])


@functools.partial(
    pl.kernel,
    out_type=jax.ShapeDtypeStruct((NC, NP, DH), jnp.float32),
    mesh=_MESH,
    compiler_params=pltpu.CompilerParams(needs_layout_passes=False, use_tc_tiling_on_sc=False),
    scratch_types=[
        pltpu.VMEM_SHARED((NP, DH), jnp.float32),
        pltpu.VMEM((NB, BW), jnp.int32),
        pltpu.VMEM((NB, BW), jnp.int32),
        pltpu.VMEM((NB, BW), jnp.float32),
        pltpu.VMEM((BW, DH), jnp.float32),
        pltpu.VMEM((BW, DH), jnp.float32),
        pltpu.VMEM((BW, DH), jnp.float32),
        pltpu.VMEM((BW, DH), jnp.float32),
        pltpu.SemaphoreType.DMA,
        pltpu.SemaphoreType.DMA,
        pltpu.SemaphoreType.DMA,
        pltpu.SemaphoreType.DMA,
    ],
)
def _sc_layer2(row_h, col_h, norm_h, h1_h, q_h,
               acc_sh, row2d, col2d, norm2d, gb0, gb1, sb0, sb1,
               gsem0, gsem1, ssem0, ssem1):
    c = lax.axis_index("c")
    s = lax.axis_index("s")
    wid = s * 2 + c
    st = s * STRIPE

    pltpu.sync_copy(row_h.at[wid], row2d)
    pltpu.sync_copy(col_h.at[wid], col2d)
    pltpu.sync_copy(norm_h.at[wid], norm2d)
    _zero_rows(gb0, BW)
    for k in range(STRIPE // BW):
        pltpu.sync_copy(gb0, acc_sh.at[pl.ds(st + k * BW, BW)])
    plsc.subcore_barrier()

    _agg_loop(row2d, col2d, norm2d, h1_h, acc_sh,
              gb0, gb1, sb0, sb1, gsem0, gsem1, ssem0, ssem1)
    plsc.subcore_barrier()
    pltpu.sync_copy(acc_sh.at[pl.ds(st, STRIPE)],
                    q_h.at[c].at[pl.ds(st, STRIPE)])


def _mm_body(x_ref, w_ref, o_ref):
    o_ref[...] = jnp.dot(x_ref[...], w_ref[...],
                         preferred_element_type=jnp.float32)


_tc_matmul = pl.pallas_call(
    _mm_body,
    out_shape=jax.ShapeDtypeStruct((NP, DH), jnp.float32),
)


def _comb_body(p_ref, dis_ref, h0_ref, o_ref):
    d2 = dis_ref[...] * dis_ref[...]
    o_ref[...] = p_ref[0] + p_ref[1] + d2 * h0_ref[...]


_tc_combine = pl.pallas_call(
    _comb_body,
    out_shape=jax.ShapeDtypeStruct((NP, DH), jnp.float32),
)


def _final_body(q_ref, h1_ref, dis_ref, w2_ref, o_ref):
    d2 = dis_ref[...] * dis_ref[...]
    h2 = q_ref[0] + q_ref[1] + d2 * h1_ref[...]
    logits = jnp.dot(h2, w2_ref[...], preferred_element_type=jnp.float32)
    m = jnp.max(logits, axis=-1, keepdims=True)
    sh = logits - m
    lse = jnp.log(jnp.sum(jnp.exp(sh), axis=-1, keepdims=True))
    o_ref[...] = sh - lse


_tc_final = pl.pallas_call(
    _final_body,
    out_shape=jax.ShapeDtypeStruct((NP, NCLS), jnp.float32),
)


def kernel(x, edge_index, edge_weight, W1, W2):
    row = edge_index[0].astype(jnp.int32)
    col = edge_index[1].astype(jnp.int32)
    pad = EP - E
    rowp = jnp.concatenate([row, jnp.zeros((pad,), jnp.int32)]).reshape(NT, NB, BW)
    colp = jnp.concatenate([col, jnp.zeros((pad,), jnp.int32)]).reshape(NT, NB, BW)
    ewp = jnp.concatenate(
        [edge_weight.astype(jnp.float32), jnp.zeros((pad,), jnp.float32)]
    ).reshape(NT, NB, BW)
    xp = jnp.pad(x.astype(jnp.float32), ((0, NP - N), (0, 0)))

    h0 = _tc_matmul(xp, W1)
    p, dis, normv = _sc_layer1(rowp, colp, ewp, h0)
    dis2d = dis.reshape(NP, 1)
    h1 = _tc_combine(p, dis2d, h0)
    q = _sc_layer2(rowp, colp, normv, h1)
    out = _tc_final(q, h1, dis2d, W2)
    return out[:N]


# speedup vs baseline: 38.0993x; 1.2391x over previous
import functools

import jax
import jax.numpy as jnp
from jax import lax
from jax.experimental import pallas as pl
from jax.experimental.pallas import tpu as pltpu
from jax.experimental.pallas import tpu_sc as plsc

N = 10000
NP = 10240
E = 320000
DF = 128
DH = 16
NCLS = 40

NC = 2
NS = 16
NT = NC * NS
BW = 128
NB = 80
EP = NT * NB * BW
STRIPE = NP // NS

_MESH = plsc.VectorSubcoreMesh(core_axis_name="c", subcore_axis_name="s")


def _splat16(v, lane):
    idx = jnp.full((16, 1), lane, jnp.int32)
    dn = lax.GatherDimensionNumbers(
        offset_dims=(), collapsed_slice_dims=(0,), start_index_map=(0,))
    return lax.gather(v, idx, dn, (1,),
                      mode=lax.GatherScatterMode.PROMISE_IN_BOUNDS)


def _zero_rows(ref, nrows):
    def body(i, _):
        ref[i, :] = jnp.zeros((DH,), jnp.float32)
        return 0
    lax.fori_loop(0, nrows, body, 0)


def _agg_loop(row2d, col2d, norm2d, tbl_h, acc_sh, gbufs, sbufs, gsems, ssems):
    for j in range(3):
        pltpu.async_copy(tbl_h.at[row2d.at[j]], gbufs[j], gsems[j])

    def body(j4, _):
        for b in range(4):
            j = j4 * 4 + b
            gb, gs = gbufs[b], gsems[b]
            sb, ss = sbufs[b], ssems[b]
            pltpu.make_async_copy(tbl_h.at[row2d.at[j]], gb, gs).wait()
            nb = (b + 3) % 4
            if b == 0:
                pltpu.async_copy(tbl_h.at[row2d.at[j + 3]], gbufs[nb], gsems[nb])
            else:
                @pl.when(j4 < NB // 4 - 1)
                def _():
                    pltpu.async_copy(tbl_h.at[row2d.at[j + 3]],
                                     gbufs[nb], gsems[nb])
            @pl.when(j4 >= 1)
            def _():
                pltpu.make_async_copy(sb, acc_sh.at[col2d.at[j]], ss).wait()

            def scale(k, _2):
                nv = norm2d[j, pl.ds(k * 16, 16)]
                for l in range(16):
                    m = k * 16 + l
                    sb[m, :] = gb[m, :] * _splat16(nv, l)
                return 0
            lax.fori_loop(0, BW // 16, scale, 0)

            pltpu.async_copy(sb, acc_sh.at[col2d.at[j]], ss, add=True)
        return 0

    lax.fori_loop(0, NB // 4, body, 0)
    for j in range(NB - 4, NB):
        b = j % 4
        pltpu.make_async_copy(sbufs[b], acc_sh.at[col2d.at[j]], ssems[b]).wait()


@functools.partial(
    pl.kernel,
    out_type=(
        jax.ShapeDtypeStruct((NC, NP, DH), jnp.float32),
        jax.ShapeDtypeStruct((NP,), jnp.float32),
        jax.ShapeDtypeStruct((NT, NB, BW), jnp.float32),
    ),
    mesh=_MESH,
    compiler_params=pltpu.CompilerParams(needs_layout_passes=False, use_tc_tiling_on_sc=False),
    scratch_types=[
        pltpu.VMEM_SHARED((NP, DH), jnp.float32),
        pltpu.VMEM_SHARED((NP,), jnp.float32),
        pltpu.VMEM_SHARED((NP,), jnp.float32),
        pltpu.VMEM((NB, BW), jnp.int32),
        pltpu.VMEM((NB, BW), jnp.int32),
        pltpu.VMEM((NB, BW), jnp.float32),
        pltpu.VMEM((8, BW), jnp.int32),
        pltpu.VMEM((8, BW), jnp.float32),
        pltpu.VMEM((STRIPE,), jnp.float32),
        pltpu.VMEM((NP,), jnp.float32),
        pltpu.VMEM((BW, DH), jnp.float32),
        pltpu.VMEM((BW, DH), jnp.float32),
        pltpu.VMEM((BW, DH), jnp.float32),
        pltpu.VMEM((BW, DH), jnp.float32),
        pltpu.VMEM((BW, DH), jnp.float32),
        pltpu.VMEM((BW, DH), jnp.float32),
        pltpu.VMEM((BW, DH), jnp.float32),
        pltpu.VMEM((BW, DH), jnp.float32),
        pltpu.SemaphoreType.DMA,
        pltpu.SemaphoreType.DMA,
        pltpu.SemaphoreType.DMA,
        pltpu.SemaphoreType.DMA,
        pltpu.SemaphoreType.DMA,
        pltpu.SemaphoreType.DMA,
        pltpu.SemaphoreType.DMA,
        pltpu.SemaphoreType.DMA,
        pltpu.SemaphoreType.DMA,
    ],
)
def _sc_layer1(row_h, col_h, ew_h, h0_h, p_h, dis_h, norm_h,
               acc_sh, deg_sh, dis_sh,
               row2d, col2d, ew2d, col_w, ew_w,
               degb, disf, gb0, gb1, gb2, gb3, sb0, sb1, sb2, sb3,
               gsem0, gsem1, gsem2, gsem3, ssem0, ssem1, ssem2, ssem3,
               hsem):
    c = lax.axis_index("c")
    s = lax.axis_index("s")
    wid = s * 2 + c
    owid = s * 2 + (1 - c)
    st = s * STRIPE

    pltpu.sync_copy(row_h.at[wid], row2d)
    pltpu.sync_copy(col_h.at[wid], col2d)
    pltpu.sync_copy(ew_h.at[wid], ew2d)
    _zero_rows(gb0, BW)
    for k in range(STRIPE // BW):
        pltpu.sync_copy(gb0, acc_sh.at[pl.ds(st + k * BW, BW)])
    def ones(i, _):
        degb[pl.ds(i * 16, 16)] = jnp.full((16,), 1.0, jnp.float32)
        return 0
    lax.fori_loop(0, STRIPE // 16, ones, 0)
    pltpu.sync_copy(degb, deg_sh.at[pl.ds(st, STRIPE)])
    plsc.subcore_barrier()

    def hist_mine(g, _):
        for i in range(8):
            j = g * 8 + i
            pltpu.async_copy(ew2d.at[j], deg_sh.at[col2d.at[j]], hsem,
                             add=True)
        for i in range(8):
            j = g * 8 + i
            pltpu.make_async_copy(ew2d.at[j], deg_sh.at[col2d.at[j]],
                                  hsem).wait()
        return 0
    lax.fori_loop(0, NB // 8, hist_mine, 0)

    def hist_part(o, _):
        pltpu.sync_copy(col_h.at[owid].at[pl.ds(o * 8, 8)], col_w)
        pltpu.sync_copy(ew_h.at[owid].at[pl.ds(o * 8, 8)], ew_w)
        for i in range(8):
            pltpu.async_copy(ew_w.at[i], deg_sh.at[col_w.at[i]], hsem,
                             add=True)
        for i in range(8):
            pltpu.make_async_copy(ew_w.at[i], deg_sh.at[col_w.at[i]],
                                  hsem).wait()
        return 0
    lax.fori_loop(0, NB // 8, hist_part, 0)
    plsc.subcore_barrier()

    pltpu.sync_copy(deg_sh.at[pl.ds(st, STRIPE)], degb)

    def newton(v, _):
        sl = pl.ds(v * 16, 16)
        dv = degb[sl]
        bits = lax.bitcast_convert_type(dv, jnp.int32)
        y = lax.bitcast_convert_type(
            jnp.full((16,), 0x5F3759DF, jnp.int32) - (bits >> 1), jnp.float32)
        half = dv * 0.5
        for _i in range(4):
            y = y * (1.5 - half * y * y)
        degb[sl] = y
        return 0
    lax.fori_loop(0, STRIPE // 16, newton, 0)
    pltpu.sync_copy(degb, dis_sh.at[pl.ds(st, STRIPE)])

    @pl.when(c == 0)
    def _():
        pltpu.sync_copy(degb, dis_h.at[pl.ds(st, STRIPE)])
    plsc.subcore_barrier()

    pltpu.sync_copy(dis_sh, disf)

    def nrm(j, _):
        def inner(k, _2):
            sl = pl.ds(k * 16, 16)
            dr = plsc.load_gather(disf, [row2d[j, sl]])
            dc = plsc.load_gather(disf, [col2d[j, sl]])
            ew2d[j, sl] = dr * ew2d[j, sl] * dc
            return 0
        lax.fori_loop(0, BW // 16, inner, 0)
        return 0
    lax.fori_loop(0, NB, nrm, 0)
    pltpu.sync_copy(ew2d, norm_h.at[wid])

    _agg_loop(row2d, col2d, ew2d, h0_h, acc_sh,
              (gb0, gb1, gb2, gb3), (sb0, sb1, sb2, sb3),
              (gsem0, gsem1, gsem2, gsem3), (ssem0, ssem1, ssem2, ssem3))
    plsc.subcore_barrier()
    pltpu.sync_copy(acc_sh.at[pl.ds(st, STRIPE)],
                    p_h.at[c].at[pl.ds(st, STRIPE)])


@functools.partial(
    pl.kernel,
    out_type=jax.ShapeDtypeStruct((NC, NP, DH), jnp.float32),
    mesh=_MESH,
    compiler_params=pltpu.CompilerParams(needs_layout_passes=False, use_tc_tiling_on_sc=False),
    scratch_types=[
        pltpu.VMEM_SHARED((NP, DH), jnp.float32),
        pltpu.VMEM((NB, BW), jnp.int32),
        pltpu.VMEM((NB, BW), jnp.int32),
        pltpu.VMEM((NB, BW), jnp.float32),
        pltpu.VMEM((BW, DH), jnp.float32),
        pltpu.VMEM((BW, DH), jnp.float32),
        pltpu.VMEM((BW, DH), jnp.float32),
        pltpu.VMEM((BW, DH), jnp.float32),
        pltpu.VMEM((BW, DH), jnp.float32),
        pltpu.VMEM((BW, DH), jnp.float32),
        pltpu.VMEM((BW, DH), jnp.float32),
        pltpu.VMEM((BW, DH), jnp.float32),
        pltpu.SemaphoreType.DMA,
        pltpu.SemaphoreType.DMA,
        pltpu.SemaphoreType.DMA,
        pltpu.SemaphoreType.DMA,
        pltpu.SemaphoreType.DMA,
        pltpu.SemaphoreType.DMA,
        pltpu.SemaphoreType.DMA,
        pltpu.SemaphoreType.DMA,
        pltpu.SemaphoreType.DMA,
    ],
)
def _sc_layer2(row_h, col_h, norm_h, h1_h, q_h,
               acc_sh, row2d, col2d, norm2d,
               gb0, gb1, gb2, gb3, sb0, sb1, sb2, sb3,
               gsem0, gsem1, gsem2, gsem3, ssem0, ssem1, ssem2, ssem3,
               hsem):
    c = lax.axis_index("c")
    s = lax.axis_index("s")
    wid = s * 2 + c
    st = s * STRIPE

    pltpu.sync_copy(row_h.at[wid], row2d)
    pltpu.sync_copy(col_h.at[wid], col2d)
    pltpu.sync_copy(norm_h.at[wid], norm2d)
    _zero_rows(gb0, BW)
    for k in range(STRIPE // BW):
        pltpu.sync_copy(gb0, acc_sh.at[pl.ds(st + k * BW, BW)])
    plsc.subcore_barrier()

    _agg_loop(row2d, col2d, norm2d, h1_h, acc_sh,
              (gb0, gb1, gb2, gb3), (sb0, sb1, sb2, sb3),
              (gsem0, gsem1, gsem2, gsem3), (ssem0, ssem1, ssem2, ssem3))
    plsc.subcore_barrier()
    pltpu.sync_copy(acc_sh.at[pl.ds(st, STRIPE)],
                    q_h.at[c].at[pl.ds(st, STRIPE)])


def _mm_body(x_ref, w_ref, o_ref):
    o_ref[...] = jnp.dot(x_ref[...], w_ref[...],
                         preferred_element_type=jnp.float32)


_tc_matmul = pl.pallas_call(
    _mm_body,
    out_shape=jax.ShapeDtypeStruct((NP, DH), jnp.float32),
)


def _comb_body(p_ref, dis_ref, h0_ref, o_ref):
    d2 = dis_ref[...] * dis_ref[...]
    o_ref[...] = p_ref[0] + p_ref[1] + d2 * h0_ref[...]


_tc_combine = pl.pallas_call(
    _comb_body,
    out_shape=jax.ShapeDtypeStruct((NP, DH), jnp.float32),
)


def _final_body(q_ref, h1_ref, dis_ref, w2_ref, o_ref):
    d2 = dis_ref[...] * dis_ref[...]
    h2 = q_ref[0] + q_ref[1] + d2 * h1_ref[...]
    logits = jnp.dot(h2, w2_ref[...], preferred_element_type=jnp.float32)
    m = jnp.max(logits, axis=-1, keepdims=True)
    sh = logits - m
    lse = jnp.log(jnp.sum(jnp.exp(sh), axis=-1, keepdims=True))
    o_ref[...] = sh - lse


_tc_final = pl.pallas_call(
    _final_body,
    out_shape=jax.ShapeDtypeStruct((NP, NCLS), jnp.float32),
)


def kernel(x, edge_index, edge_weight, W1, W2):
    row = edge_index[0].astype(jnp.int32)
    col = edge_index[1].astype(jnp.int32)
    pad = EP - E
    rowp = jnp.concatenate([row, jnp.zeros((pad,), jnp.int32)]).reshape(NT, NB, BW)
    colp = jnp.concatenate([col, jnp.zeros((pad,), jnp.int32)]).reshape(NT, NB, BW)
    ewp = jnp.concatenate(
        [edge_weight.astype(jnp.float32), jnp.zeros((pad,), jnp.float32)]
    ).reshape(NT, NB, BW)
    xp = jnp.pad(x.astype(jnp.float32), ((0, NP - N), (0, 0)))

    h0 = _tc_matmul(xp, W1)
    p, dis, normv = _sc_layer1(rowp, colp, ewp, h0)
    dis2d = dis.reshape(NP, 1)
    h1 = _tc_combine(p, dis2d, h0)
    q = _sc_layer2(rowp, colp, normv, h1)
    out = _tc_final(q, h1, dis2d, W2)
    return out[:N]


# speedup vs baseline: 38.1121x; 1.0003x over previous
import functools

import jax
import jax.numpy as jnp
from jax import lax
from jax.experimental import pallas as pl
from jax.experimental.pallas import tpu as pltpu
from jax.experimental.pallas import tpu_sc as plsc

N = 10000
NP = 10240
E = 320000
DF = 128
DH = 16
NCLS = 40

NC = 2
NS = 16
NT = NC * NS
BW = 128
NB = 80
EP = NT * NB * BW
STRIPE = NP // NS

_MESH = plsc.VectorSubcoreMesh(core_axis_name="c", subcore_axis_name="s")


def _splat16(v, lane):
    idx = jnp.full((16, 1), lane, jnp.int32)
    dn = lax.GatherDimensionNumbers(
        offset_dims=(), collapsed_slice_dims=(0,), start_index_map=(0,))
    return lax.gather(v, idx, dn, (1,),
                      mode=lax.GatherScatterMode.PROMISE_IN_BOUNDS)


def _zero_rows(ref, nrows):
    def body(i, _):
        ref[i, :] = jnp.zeros((DH,), jnp.float32)
        return 0
    lax.fori_loop(0, nrows, body, 0)


def _agg_loop(row2d, col2d, norm2d, tbl_h, acc_sh, gbufs, sbufs, gsems, ssems):
    for j in range(3):
        pltpu.async_copy(tbl_h.at[row2d.at[j]], gbufs[j], gsems[j])

    def body(j4, _):
        for b in range(4):
            j = j4 * 4 + b
            gb, gs = gbufs[b], gsems[b]
            sb, ss = sbufs[b], ssems[b]
            pltpu.make_async_copy(tbl_h.at[row2d.at[j]], gb, gs).wait()
            nb = (b + 3) % 4
            if b == 0:
                pltpu.async_copy(tbl_h.at[row2d.at[j + 3]], gbufs[nb], gsems[nb])
            else:
                @pl.when(j4 < NB // 4 - 1)
                def _():
                    pltpu.async_copy(tbl_h.at[row2d.at[j + 3]],
                                     gbufs[nb], gsems[nb])
            @pl.when(j4 >= 1)
            def _():
                pltpu.make_async_copy(sb, acc_sh.at[col2d.at[j]], ss).wait()

            def scale(k, _2):
                nv = norm2d[j, pl.ds(k * 16, 16)]
                for l in range(16):
                    m = k * 16 + l
                    sb[m, :] = gb[m, :] * nv[l]
                return 0
            lax.fori_loop(0, BW // 16, scale, 0)

            pltpu.async_copy(sb, acc_sh.at[col2d.at[j]], ss, add=True)
        return 0

    lax.fori_loop(0, NB // 4, body, 0)
    for j in range(NB - 4, NB):
        b = j % 4
        pltpu.make_async_copy(sbufs[b], acc_sh.at[col2d.at[j]], ssems[b]).wait()


@functools.partial(
    pl.kernel,
    out_type=(
        jax.ShapeDtypeStruct((NC, NP, DH), jnp.float32),
        jax.ShapeDtypeStruct((NP,), jnp.float32),
        jax.ShapeDtypeStruct((NT, NB, BW), jnp.float32),
    ),
    mesh=_MESH,
    compiler_params=pltpu.CompilerParams(needs_layout_passes=False, use_tc_tiling_on_sc=False),
    scratch_types=[
        pltpu.VMEM_SHARED((NP, DH), jnp.float32),
        pltpu.VMEM_SHARED((NP,), jnp.float32),
        pltpu.VMEM_SHARED((NP,), jnp.float32),
        pltpu.VMEM((NB, BW), jnp.int32),
        pltpu.VMEM((NB, BW), jnp.int32),
        pltpu.VMEM((NB, BW), jnp.float32),
        pltpu.VMEM((8, BW), jnp.int32),
        pltpu.VMEM((8, BW), jnp.float32),
        pltpu.VMEM((STRIPE,), jnp.float32),
        pltpu.VMEM((NP,), jnp.float32),
        pltpu.VMEM((BW, DH), jnp.float32),
        pltpu.VMEM((BW, DH), jnp.float32),
        pltpu.VMEM((BW, DH), jnp.float32),
        pltpu.VMEM((BW, DH), jnp.float32),
        pltpu.VMEM((BW, DH), jnp.float32),
        pltpu.VMEM((BW, DH), jnp.float32),
        pltpu.VMEM((BW, DH), jnp.float32),
        pltpu.VMEM((BW, DH), jnp.float32),
        pltpu.SemaphoreType.DMA,
        pltpu.SemaphoreType.DMA,
        pltpu.SemaphoreType.DMA,
        pltpu.SemaphoreType.DMA,
        pltpu.SemaphoreType.DMA,
        pltpu.SemaphoreType.DMA,
        pltpu.SemaphoreType.DMA,
        pltpu.SemaphoreType.DMA,
        pltpu.SemaphoreType.DMA,
    ],
)
def _sc_layer1(row_h, col_h, ew_h, h0_h, p_h, dis_h, norm_h,
               acc_sh, deg_sh, dis_sh,
               row2d, col2d, ew2d, col_w, ew_w,
               degb, disf, gb0, gb1, gb2, gb3, sb0, sb1, sb2, sb3,
               gsem0, gsem1, gsem2, gsem3, ssem0, ssem1, ssem2, ssem3,
               hsem):
    c = lax.axis_index("c")
    s = lax.axis_index("s")
    wid = s * 2 + c
    owid = s * 2 + (1 - c)
    st = s * STRIPE

    pltpu.sync_copy(row_h.at[wid], row2d)
    pltpu.sync_copy(col_h.at[wid], col2d)
    pltpu.sync_copy(ew_h.at[wid], ew2d)
    _zero_rows(gb0, BW)
    for k in range(STRIPE // BW):
        pltpu.sync_copy(gb0, acc_sh.at[pl.ds(st + k * BW, BW)])
    def ones(i, _):
        degb[pl.ds(i * 16, 16)] = jnp.full((16,), 1.0, jnp.float32)
        return 0
    lax.fori_loop(0, STRIPE // 16, ones, 0)
    pltpu.sync_copy(degb, deg_sh.at[pl.ds(st, STRIPE)])
    plsc.subcore_barrier()

    def hist_mine(g, _):
        for i in range(8):
            j = g * 8 + i
            pltpu.async_copy(ew2d.at[j], deg_sh.at[col2d.at[j]], hsem,
                             add=True)
        for i in range(8):
            j = g * 8 + i
            pltpu.make_async_copy(ew2d.at[j], deg_sh.at[col2d.at[j]],
                                  hsem).wait()
        return 0
    lax.fori_loop(0, NB // 8, hist_mine, 0)

    def hist_part(o, _):
        pltpu.sync_copy(col_h.at[owid].at[pl.ds(o * 8, 8)], col_w)
        pltpu.sync_copy(ew_h.at[owid].at[pl.ds(o * 8, 8)], ew_w)
        for i in range(8):
            pltpu.async_copy(ew_w.at[i], deg_sh.at[col_w.at[i]], hsem,
                             add=True)
        for i in range(8):
            pltpu.make_async_copy(ew_w.at[i], deg_sh.at[col_w.at[i]],
                                  hsem).wait()
        return 0
    lax.fori_loop(0, NB // 8, hist_part, 0)
    plsc.subcore_barrier()

    pltpu.sync_copy(deg_sh.at[pl.ds(st, STRIPE)], degb)

    def newton(v, _):
        sl = pl.ds(v * 16, 16)
        dv = degb[sl]
        bits = lax.bitcast_convert_type(dv, jnp.int32)
        y = lax.bitcast_convert_type(
            jnp.full((16,), 0x5F3759DF, jnp.int32) - (bits >> 1), jnp.float32)
        half = dv * 0.5
        for _i in range(4):
            y = y * (1.5 - half * y * y)
        degb[sl] = y
        return 0
    lax.fori_loop(0, STRIPE // 16, newton, 0)
    pltpu.sync_copy(degb, dis_sh.at[pl.ds(st, STRIPE)])

    @pl.when(c == 0)
    def _():
        pltpu.sync_copy(degb, dis_h.at[pl.ds(st, STRIPE)])
    plsc.subcore_barrier()

    pltpu.sync_copy(dis_sh, disf)

    def nrm(j, _):
        def inner(k, _2):
            sl = pl.ds(k * 16, 16)
            dr = plsc.load_gather(disf, [row2d[j, sl]])
            dc = plsc.load_gather(disf, [col2d[j, sl]])
            ew2d[j, sl] = dr * ew2d[j, sl] * dc
            return 0
        lax.fori_loop(0, BW // 16, inner, 0)
        return 0
    lax.fori_loop(0, NB, nrm, 0)
    pltpu.sync_copy(ew2d, norm_h.at[wid])

    _agg_loop(row2d, col2d, ew2d, h0_h, acc_sh,
              (gb0, gb1, gb2, gb3), (sb0, sb1, sb2, sb3),
              (gsem0, gsem1, gsem2, gsem3), (ssem0, ssem1, ssem2, ssem3))
    plsc.subcore_barrier()
    pltpu.sync_copy(acc_sh.at[pl.ds(st, STRIPE)],
                    p_h.at[c].at[pl.ds(st, STRIPE)])


@functools.partial(
    pl.kernel,
    out_type=jax.ShapeDtypeStruct((NC, NP, DH), jnp.float32),
    mesh=_MESH,
    compiler_params=pltpu.CompilerParams(needs_layout_passes=False, use_tc_tiling_on_sc=False),
    scratch_types=[
        pltpu.VMEM_SHARED((NP, DH), jnp.float32),
        pltpu.VMEM((NB, BW), jnp.int32),
        pltpu.VMEM((NB, BW), jnp.int32),
        pltpu.VMEM((NB, BW), jnp.float32),
        pltpu.VMEM((BW, DH), jnp.float32),
        pltpu.VMEM((BW, DH), jnp.float32),
        pltpu.VMEM((BW, DH), jnp.float32),
        pltpu.VMEM((BW, DH), jnp.float32),
        pltpu.VMEM((BW, DH), jnp.float32),
        pltpu.VMEM((BW, DH), jnp.float32),
        pltpu.VMEM((BW, DH), jnp.float32),
        pltpu.VMEM((BW, DH), jnp.float32),
        pltpu.SemaphoreType.DMA,
        pltpu.SemaphoreType.DMA,
        pltpu.SemaphoreType.DMA,
        pltpu.SemaphoreType.DMA,
        pltpu.SemaphoreType.DMA,
        pltpu.SemaphoreType.DMA,
        pltpu.SemaphoreType.DMA,
        pltpu.SemaphoreType.DMA,
        pltpu.SemaphoreType.DMA,
    ],
)
def _sc_layer2(row_h, col_h, norm_h, h1_h, q_h,
               acc_sh, row2d, col2d, norm2d,
               gb0, gb1, gb2, gb3, sb0, sb1, sb2, sb3,
               gsem0, gsem1, gsem2, gsem3, ssem0, ssem1, ssem2, ssem3,
               hsem):
    c = lax.axis_index("c")
    s = lax.axis_index("s")
    wid = s * 2 + c
    st = s * STRIPE

    pltpu.sync_copy(row_h.at[wid], row2d)
    pltpu.sync_copy(col_h.at[wid], col2d)
    pltpu.sync_copy(norm_h.at[wid], norm2d)
    _zero_rows(gb0, BW)
    for k in range(STRIPE // BW):
        pltpu.sync_copy(gb0, acc_sh.at[pl.ds(st + k * BW, BW)])
    plsc.subcore_barrier()

    _agg_loop(row2d, col2d, norm2d, h1_h, acc_sh,
              (gb0, gb1, gb2, gb3), (sb0, sb1, sb2, sb3),
              (gsem0, gsem1, gsem2, gsem3), (ssem0, ssem1, ssem2, ssem3))
    plsc.subcore_barrier()
    pltpu.sync_copy(acc_sh.at[pl.ds(st, STRIPE)],
                    q_h.at[c].at[pl.ds(st, STRIPE)])


def _mm_body(x_ref, w_ref, o_ref):
    o_ref[...] = jnp.dot(x_ref[...], w_ref[...],
                         preferred_element_type=jnp.float32)


_tc_matmul = pl.pallas_call(
    _mm_body,
    out_shape=jax.ShapeDtypeStruct((NP, DH), jnp.float32),
)


def _comb_body(p_ref, dis_ref, h0_ref, o_ref):
    d2 = dis_ref[...] * dis_ref[...]
    o_ref[...] = p_ref[0] + p_ref[1] + d2 * h0_ref[...]


_tc_combine = pl.pallas_call(
    _comb_body,
    out_shape=jax.ShapeDtypeStruct((NP, DH), jnp.float32),
)


def _final_body(q_ref, h1_ref, dis_ref, w2_ref, o_ref):
    d2 = dis_ref[...] * dis_ref[...]
    h2 = q_ref[0] + q_ref[1] + d2 * h1_ref[...]
    logits = jnp.dot(h2, w2_ref[...], preferred_element_type=jnp.float32)
    m = jnp.max(logits, axis=-1, keepdims=True)
    sh = logits - m
    lse = jnp.log(jnp.sum(jnp.exp(sh), axis=-1, keepdims=True))
    o_ref[...] = sh - lse


_tc_final = pl.pallas_call(
    _final_body,
    out_shape=jax.ShapeDtypeStruct((NP, NCLS), jnp.float32),
)


def kernel(x, edge_index, edge_weight, W1, W2):
    row = edge_index[0].astype(jnp.int32)
    col = edge_index[1].astype(jnp.int32)
    pad = EP - E
    rowp = jnp.concatenate([row, jnp.zeros((pad,), jnp.int32)]).reshape(NT, NB, BW)
    colp = jnp.concatenate([col, jnp.zeros((pad,), jnp.int32)]).reshape(NT, NB, BW)
    ewp = jnp.concatenate(
        [edge_weight.astype(jnp.float32), jnp.zeros((pad,), jnp.float32)]
    ).reshape(NT, NB, BW)
    xp = jnp.pad(x.astype(jnp.float32), ((0, NP - N), (0, 0)))

    h0 = _tc_matmul(xp, W1)
    p, dis, normv = _sc_layer1(rowp, colp, ewp, h0)
    dis2d = dis.reshape(NP, 1)
    h1 = _tc_combine(p, dis2d, h0)
    q = _sc_layer2(rowp, colp, normv, h1)
    out = _tc_final(q, h1, dis2d, W2)
    return out[:N]


# speedup vs baseline: 43.8349x; 1.1502x over previous
import functools

import jax
import jax.numpy as jnp
from jax import lax
from jax.experimental import pallas as pl
from jax.experimental.pallas import tpu as pltpu
from jax.experimental.pallas import tpu_sc as plsc

N = 10000
NP = 10240
E = 320000
DF = 128
DH = 16
NCLS = 40

NC = 2
NS = 16
NT = NC * NS
BW = 128
NB = 80
EP = NT * NB * BW
STRIPE = NP // NS

_MESH = plsc.VectorSubcoreMesh(core_axis_name="c", subcore_axis_name="s")


def _splat16(v, lane):
    idx = jnp.full((16, 1), lane, jnp.int32)
    dn = lax.GatherDimensionNumbers(
        offset_dims=(), collapsed_slice_dims=(0,), start_index_map=(0,))
    return lax.gather(v, idx, dn, (1,),
                      mode=lax.GatherScatterMode.PROMISE_IN_BOUNDS)


def _zero_rows(ref, nrows):
    def body(i, _):
        ref[i, :] = jnp.zeros((DH,), jnp.float32)
        return 0
    lax.fori_loop(0, nrows, body, 0)


def _agg_loop(row2d, col2d, norm2d, tbl_h, acc_sh, gbufs, sbufs, gsems, ssems):
    for j in range(3):
        pltpu.async_copy(tbl_h.at[row2d.at[j]], gbufs[j], gsems[j])

    def body(j4, _):
        for b in range(4):
            j = j4 * 4 + b
            gb, gs = gbufs[b], gsems[b]
            sb, ss = sbufs[b], ssems[b]
            pltpu.make_async_copy(tbl_h.at[row2d.at[j]], gb, gs).wait()
            nb = (b + 3) % 4
            if b == 0:
                pltpu.async_copy(tbl_h.at[row2d.at[j + 3]], gbufs[nb], gsems[nb])
            else:
                @pl.when(j4 < NB // 4 - 1)
                def _():
                    pltpu.async_copy(tbl_h.at[row2d.at[j + 3]],
                                     gbufs[nb], gsems[nb])
            @pl.when(j4 >= 1)
            def _():
                pltpu.make_async_copy(sb, acc_sh.at[col2d.at[j]], ss).wait()

            def scale(k, _2):
                nv = norm2d[j, pl.ds(k * 16, 16)]
                for l in range(16):
                    m = k * 16 + l
                    sb[m, :] = gb[m, :] * nv[l]
                return 0
            lax.fori_loop(0, BW // 16, scale, 0)

            pltpu.async_copy(sb, acc_sh.at[col2d.at[j]], ss, add=True)
        return 0

    lax.fori_loop(0, NB // 4, body, 0)
    for j in range(NB - 4, NB):
        b = j % 4
        pltpu.make_async_copy(sbufs[b], acc_sh.at[col2d.at[j]], ssems[b]).wait()


@functools.partial(
    pl.kernel,
    out_type=(
        jax.ShapeDtypeStruct((NC, NP, DH), jnp.float32),
        jax.ShapeDtypeStruct((NP,), jnp.float32),
        jax.ShapeDtypeStruct((NT, NB, BW), jnp.float32),
    ),
    mesh=_MESH,
    compiler_params=pltpu.CompilerParams(needs_layout_passes=False, use_tc_tiling_on_sc=False),
    scratch_types=[
        pltpu.VMEM_SHARED((NP, DH), jnp.float32),
        pltpu.VMEM_SHARED((NP,), jnp.float32),
        pltpu.VMEM_SHARED((NP,), jnp.float32),
        pltpu.VMEM((NB, BW), jnp.int32),
        pltpu.VMEM((NB, BW), jnp.int32),
        pltpu.VMEM((NB, BW), jnp.float32),
        pltpu.VMEM((8, BW), jnp.int32),
        pltpu.VMEM((8, BW), jnp.float32),
        pltpu.VMEM((STRIPE,), jnp.float32),
        pltpu.VMEM((NP,), jnp.float32),
        pltpu.VMEM((BW, DH), jnp.float32),
        pltpu.VMEM((BW, DH), jnp.float32),
        pltpu.VMEM((BW, DH), jnp.float32),
        pltpu.VMEM((BW, DH), jnp.float32),
        pltpu.VMEM((BW, DH), jnp.float32),
        pltpu.VMEM((BW, DH), jnp.float32),
        pltpu.VMEM((BW, DH), jnp.float32),
        pltpu.VMEM((BW, DH), jnp.float32),
        pltpu.SemaphoreType.DMA,
        pltpu.SemaphoreType.DMA,
        pltpu.SemaphoreType.DMA,
        pltpu.SemaphoreType.DMA,
        pltpu.SemaphoreType.DMA,
        pltpu.SemaphoreType.DMA,
        pltpu.SemaphoreType.DMA,
        pltpu.SemaphoreType.DMA,
        pltpu.SemaphoreType.DMA,
    ],
)
def _sc_layer1(row_h, col_h, ew_h, h0_h, p_h, dis_h, norm_h,
               acc_sh, deg_sh, dis_sh,
               row2d, col2d, ew2d, col_w, ew_w,
               degb, disf, gb0, gb1, gb2, gb3, sb0, sb1, sb2, sb3,
               gsem0, gsem1, gsem2, gsem3, ssem0, ssem1, ssem2, ssem3,
               hsem):
    c = lax.axis_index("c")
    s = lax.axis_index("s")
    wid = s * 2 + c
    owid = s * 2 + (1 - c)
    st = s * STRIPE

    pltpu.sync_copy(row_h.at[wid], row2d)
    pltpu.sync_copy(col_h.at[wid], col2d)
    pltpu.sync_copy(ew_h.at[wid], ew2d)
    _zero_rows(gb0, BW)
    for k in range(STRIPE // BW):
        pltpu.sync_copy(gb0, acc_sh.at[pl.ds(st + k * BW, BW)])
    def ones(i, _):
        degb[pl.ds(i * 16, 16)] = jnp.full((16,), 1.0, jnp.float32)
        return 0
    lax.fori_loop(0, STRIPE // 16, ones, 0)
    pltpu.sync_copy(degb, deg_sh.at[pl.ds(st, STRIPE)])
    plsc.subcore_barrier()

    def hist_mine(g, _):
        for i in range(8):
            j = g * 8 + i
            pltpu.async_copy(ew2d.at[j], deg_sh.at[col2d.at[j]], hsem,
                             add=True)
        for i in range(8):
            j = g * 8 + i
            pltpu.make_async_copy(ew2d.at[j], deg_sh.at[col2d.at[j]],
                                  hsem).wait()
        return 0
    lax.fori_loop(0, NB // 8, hist_mine, 0)

    def hist_part(o, _):
        pltpu.sync_copy(col_h.at[owid].at[pl.ds(o * 8, 8)], col_w)
        pltpu.sync_copy(ew_h.at[owid].at[pl.ds(o * 8, 8)], ew_w)
        for i in range(8):
            pltpu.async_copy(ew_w.at[i], deg_sh.at[col_w.at[i]], hsem,
                             add=True)
        for i in range(8):
            pltpu.make_async_copy(ew_w.at[i], deg_sh.at[col_w.at[i]],
                                  hsem).wait()
        return 0
    lax.fori_loop(0, NB // 8, hist_part, 0)
    plsc.subcore_barrier()

    pltpu.sync_copy(deg_sh.at[pl.ds(st, STRIPE)], degb)

    def newton(v, _):
        sl = pl.ds(v * 16, 16)
        dv = degb[sl]
        bits = lax.bitcast_convert_type(dv, jnp.int32)
        y = lax.bitcast_convert_type(
            jnp.full((16,), 0x5F3759DF, jnp.int32) - (bits >> 1), jnp.float32)
        half = dv * 0.5
        for _i in range(4):
            y = y * (1.5 - half * y * y)
        degb[sl] = y
        return 0
    lax.fori_loop(0, STRIPE // 16, newton, 0)
    pltpu.sync_copy(degb, dis_sh.at[pl.ds(st, STRIPE)])

    @pl.when(c == 0)
    def _():
        pltpu.sync_copy(degb, dis_h.at[pl.ds(st, STRIPE)])
    plsc.subcore_barrier()

    pltpu.sync_copy(dis_sh, disf)

    def nrm(j, _):
        def inner(k, _2):
            sl = pl.ds(k * 16, 16)
            dr = plsc.load_gather(disf, [row2d[j, sl]])
            dc = plsc.load_gather(disf, [col2d[j, sl]])
            ew2d[j, sl] = dr * ew2d[j, sl] * dc
            return 0
        lax.fori_loop(0, BW // 16, inner, 0)
        return 0
    lax.fori_loop(0, NB, nrm, 0)
    pltpu.sync_copy(ew2d, norm_h.at[wid])

    _agg_loop(row2d, col2d, ew2d, h0_h, acc_sh,
              (gb0, gb1, gb2, gb3), (sb0, sb1, sb2, sb3),
              (gsem0, gsem1, gsem2, gsem3), (ssem0, ssem1, ssem2, ssem3))
    plsc.subcore_barrier()
    pltpu.sync_copy(acc_sh.at[pl.ds(st, STRIPE)],
                    p_h.at[c].at[pl.ds(st, STRIPE)])


@functools.partial(
    pl.kernel,
    out_type=jax.ShapeDtypeStruct((NC, NP, DH), jnp.float32),
    mesh=_MESH,
    compiler_params=pltpu.CompilerParams(needs_layout_passes=False, use_tc_tiling_on_sc=False),
    scratch_types=[
        pltpu.VMEM_SHARED((NP, DH), jnp.float32),
        pltpu.VMEM_SHARED((NP, DH), jnp.float32),
        pltpu.VMEM((NB, BW), jnp.int32),
        pltpu.VMEM((NB, BW), jnp.int32),
        pltpu.VMEM((NB, BW), jnp.float32),
        pltpu.VMEM((BW, DH), jnp.float32),
        pltpu.VMEM((BW, DH), jnp.float32),
        pltpu.VMEM((BW, DH), jnp.float32),
        pltpu.VMEM((BW, DH), jnp.float32),
        pltpu.VMEM((BW, DH), jnp.float32),
        pltpu.VMEM((BW, DH), jnp.float32),
        pltpu.VMEM((BW, DH), jnp.float32),
        pltpu.VMEM((BW, DH), jnp.float32),
        pltpu.SemaphoreType.DMA,
        pltpu.SemaphoreType.DMA,
        pltpu.SemaphoreType.DMA,
        pltpu.SemaphoreType.DMA,
        pltpu.SemaphoreType.DMA,
        pltpu.SemaphoreType.DMA,
        pltpu.SemaphoreType.DMA,
        pltpu.SemaphoreType.DMA,
        pltpu.SemaphoreType.DMA,
    ],
)
def _sc_layer2(row_h, col_h, norm_h, h1_h, q_h,
               acc_sh, h_sh, row2d, col2d, norm2d,
               gb0, gb1, gb2, gb3, sb0, sb1, sb2, sb3,
               gsem0, gsem1, gsem2, gsem3, ssem0, ssem1, ssem2, ssem3,
               hsem):
    c = lax.axis_index("c")
    s = lax.axis_index("s")
    wid = s * 2 + c
    st = s * STRIPE

    pltpu.sync_copy(row_h.at[wid], row2d)
    pltpu.sync_copy(col_h.at[wid], col2d)
    pltpu.sync_copy(norm_h.at[wid], norm2d)
    pltpu.sync_copy(h1_h.at[pl.ds(st, STRIPE)], h_sh.at[pl.ds(st, STRIPE)])
    _zero_rows(gb0, BW)
    for k in range(STRIPE // BW):
        pltpu.sync_copy(gb0, acc_sh.at[pl.ds(st + k * BW, BW)])
    plsc.subcore_barrier()

    _agg_loop(row2d, col2d, norm2d, h_sh, acc_sh,
              (gb0, gb1, gb2, gb3), (sb0, sb1, sb2, sb3),
              (gsem0, gsem1, gsem2, gsem3), (ssem0, ssem1, ssem2, ssem3))
    plsc.subcore_barrier()
    pltpu.sync_copy(acc_sh.at[pl.ds(st, STRIPE)],
                    q_h.at[c].at[pl.ds(st, STRIPE)])


def _mm_body(x_ref, w_ref, o_ref):
    o_ref[...] = jnp.dot(x_ref[...], w_ref[...],
                         preferred_element_type=jnp.float32)


_tc_matmul = pl.pallas_call(
    _mm_body,
    out_shape=jax.ShapeDtypeStruct((NP, DH), jnp.float32),
)


def _comb_body(p_ref, dis_ref, h0_ref, o_ref):
    d2 = dis_ref[...] * dis_ref[...]
    o_ref[...] = p_ref[0] + p_ref[1] + d2 * h0_ref[...]


_tc_combine = pl.pallas_call(
    _comb_body,
    out_shape=jax.ShapeDtypeStruct((NP, DH), jnp.float32),
)


def _final_body(q_ref, h1_ref, dis_ref, w2_ref, o_ref):
    d2 = dis_ref[...] * dis_ref[...]
    h2 = q_ref[0] + q_ref[1] + d2 * h1_ref[...]
    logits = jnp.dot(h2, w2_ref[...], preferred_element_type=jnp.float32)
    m = jnp.max(logits, axis=-1, keepdims=True)
    sh = logits - m
    lse = jnp.log(jnp.sum(jnp.exp(sh), axis=-1, keepdims=True))
    o_ref[...] = sh - lse


_tc_final = pl.pallas_call(
    _final_body,
    out_shape=jax.ShapeDtypeStruct((NP, NCLS), jnp.float32),
)


def kernel(x, edge_index, edge_weight, W1, W2):
    row = edge_index[0].astype(jnp.int32)
    col = edge_index[1].astype(jnp.int32)
    pad = EP - E
    rowp = jnp.concatenate([row, jnp.zeros((pad,), jnp.int32)]).reshape(NT, NB, BW)
    colp = jnp.concatenate([col, jnp.zeros((pad,), jnp.int32)]).reshape(NT, NB, BW)
    ewp = jnp.concatenate(
        [edge_weight.astype(jnp.float32), jnp.zeros((pad,), jnp.float32)]
    ).reshape(NT, NB, BW)
    xp = jnp.pad(x.astype(jnp.float32), ((0, NP - N), (0, 0)))

    h0 = _tc_matmul(xp, W1)
    p, dis, normv = _sc_layer1(rowp, colp, ewp, h0)
    dis2d = dis.reshape(NP, 1)
    h1 = _tc_combine(p, dis2d, h0)
    q = _sc_layer2(rowp, colp, normv, h1)
    out = _tc_final(q, h1, dis2d, W2)
    return out[:N]


# speedup vs baseline: 53.5473x; 1.2216x over previous
import functools

import jax
import jax.numpy as jnp
from jax import lax
from jax.experimental import pallas as pl
from jax.experimental.pallas import tpu as pltpu
from jax.experimental.pallas import tpu_sc as plsc

N = 10000
NP = 10240
E = 320000
DF = 128
DH = 16
NCLS = 40

NC = 2
NS = 16
NT = NC * NS
BW = 128
NB = 80
EP = NT * NB * BW
STRIPE = NP // NS

_MESH = plsc.VectorSubcoreMesh(core_axis_name="c", subcore_axis_name="s")


def _splat16(v, lane):
    idx = jnp.full((16, 1), lane, jnp.int32)
    dn = lax.GatherDimensionNumbers(
        offset_dims=(), collapsed_slice_dims=(0,), start_index_map=(0,))
    return lax.gather(v, idx, dn, (1,),
                      mode=lax.GatherScatterMode.PROMISE_IN_BOUNDS)


def _zero_rows(ref, nrows):
    def body(i, _):
        ref[i, :] = jnp.zeros((DH,), jnp.float32)
        return 0
    lax.fori_loop(0, nrows, body, 0)


def _agg_loop(row2d, col2d, norm2d, tbl_h, acc_sh, gbufs, sbufs, gsems, ssems):
    for j in range(3):
        pltpu.async_copy(tbl_h.at[row2d.at[j]], gbufs[j], gsems[j])

    def body(j4, _):
        for b in range(4):
            j = j4 * 4 + b
            gb, gs = gbufs[b], gsems[b]
            sb, ss = sbufs[b], ssems[b]
            pltpu.make_async_copy(tbl_h.at[row2d.at[j]], gb, gs).wait()
            nb = (b + 3) % 4
            if b == 0:
                pltpu.async_copy(tbl_h.at[row2d.at[j + 3]], gbufs[nb], gsems[nb])
            else:
                @pl.when(j4 < NB // 4 - 1)
                def _():
                    pltpu.async_copy(tbl_h.at[row2d.at[j + 3]],
                                     gbufs[nb], gsems[nb])
            @pl.when(j4 >= 1)
            def _():
                pltpu.make_async_copy(sb, acc_sh.at[col2d.at[j]], ss).wait()

            def scale(k, _2):
                nv = norm2d[j, pl.ds(k * 16, 16)]
                for l in range(16):
                    m = k * 16 + l
                    sb[m, :] = gb[m, :] * nv[l]
                return 0
            lax.fori_loop(0, BW // 16, scale, 0)

            pltpu.async_copy(sb, acc_sh.at[col2d.at[j]], ss, add=True)
        return 0

    lax.fori_loop(0, NB // 4, body, 0)
    for j in range(NB - 4, NB):
        b = j % 4
        pltpu.make_async_copy(sbufs[b], acc_sh.at[col2d.at[j]], ssems[b]).wait()


@functools.partial(
    pl.kernel,
    out_type=(
        jax.ShapeDtypeStruct((NC, NP, DH), jnp.float32),
        jax.ShapeDtypeStruct((NP,), jnp.float32),
        jax.ShapeDtypeStruct((NT, NB, BW), jnp.float32),
    ),
    mesh=_MESH,
    compiler_params=pltpu.CompilerParams(needs_layout_passes=False, use_tc_tiling_on_sc=False),
    scratch_types=[
        pltpu.VMEM_SHARED((NP, DH), jnp.float32),
        pltpu.VMEM_SHARED((NP, DH), jnp.float32),
        pltpu.VMEM_SHARED((NP,), jnp.float32),
        pltpu.VMEM_SHARED((NP,), jnp.float32),
        pltpu.VMEM((NB, BW), jnp.int32),
        pltpu.VMEM((NB, BW), jnp.int32),
        pltpu.VMEM((NB, BW), jnp.float32),
        pltpu.VMEM((8, BW), jnp.int32),
        pltpu.VMEM((8, BW), jnp.float32),
        pltpu.VMEM((STRIPE,), jnp.float32),
        pltpu.VMEM((NP,), jnp.float32),
        pltpu.VMEM((BW, DH), jnp.float32),
        pltpu.VMEM((BW, DH), jnp.float32),
        pltpu.VMEM((BW, DH), jnp.float32),
        pltpu.VMEM((BW, DH), jnp.float32),
        pltpu.VMEM((BW, DH), jnp.float32),
        pltpu.VMEM((BW, DH), jnp.float32),
        pltpu.VMEM((BW, DH), jnp.float32),
        pltpu.VMEM((BW, DH), jnp.float32),
        pltpu.SemaphoreType.DMA,
        pltpu.SemaphoreType.DMA,
        pltpu.SemaphoreType.DMA,
        pltpu.SemaphoreType.DMA,
        pltpu.SemaphoreType.DMA,
        pltpu.SemaphoreType.DMA,
        pltpu.SemaphoreType.DMA,
        pltpu.SemaphoreType.DMA,
        pltpu.SemaphoreType.DMA,
    ],
)
def _sc_layer1(row_h, col_h, ew_h, h0_h, p_h, dis_h, norm_h,
               acc_sh, h_sh, deg_sh, dis_sh,
               row2d, col2d, ew2d, col_w, ew_w,
               degb, disf, gb0, gb1, gb2, gb3, sb0, sb1, sb2, sb3,
               gsem0, gsem1, gsem2, gsem3, ssem0, ssem1, ssem2, ssem3,
               hsem):
    c = lax.axis_index("c")
    s = lax.axis_index("s")
    wid = s * 2 + c
    owid = s * 2 + (1 - c)
    st = s * STRIPE

    pltpu.sync_copy(row_h.at[wid], row2d)
    pltpu.sync_copy(col_h.at[wid], col2d)
    pltpu.sync_copy(ew_h.at[wid], ew2d)
    pltpu.sync_copy(h0_h.at[pl.ds(st, STRIPE)], h_sh.at[pl.ds(st, STRIPE)])
    _zero_rows(gb0, BW)
    for k in range(STRIPE // BW):
        pltpu.sync_copy(gb0, acc_sh.at[pl.ds(st + k * BW, BW)])
    def ones(i, _):
        degb[pl.ds(i * 16, 16)] = jnp.full((16,), 1.0, jnp.float32)
        return 0
    lax.fori_loop(0, STRIPE // 16, ones, 0)
    pltpu.sync_copy(degb, deg_sh.at[pl.ds(st, STRIPE)])
    plsc.subcore_barrier()

    def hist_mine(g, _):
        for i in range(8):
            j = g * 8 + i
            pltpu.async_copy(ew2d.at[j], deg_sh.at[col2d.at[j]], hsem,
                             add=True)
        for i in range(8):
            j = g * 8 + i
            pltpu.make_async_copy(ew2d.at[j], deg_sh.at[col2d.at[j]],
                                  hsem).wait()
        return 0
    lax.fori_loop(0, NB // 8, hist_mine, 0)

    def hist_part(o, _):
        pltpu.sync_copy(col_h.at[owid].at[pl.ds(o * 8, 8)], col_w)
        pltpu.sync_copy(ew_h.at[owid].at[pl.ds(o * 8, 8)], ew_w)
        for i in range(8):
            pltpu.async_copy(ew_w.at[i], deg_sh.at[col_w.at[i]], hsem,
                             add=True)
        for i in range(8):
            pltpu.make_async_copy(ew_w.at[i], deg_sh.at[col_w.at[i]],
                                  hsem).wait()
        return 0
    lax.fori_loop(0, NB // 8, hist_part, 0)
    plsc.subcore_barrier()

    pltpu.sync_copy(deg_sh.at[pl.ds(st, STRIPE)], degb)

    def newton(v, _):
        sl = pl.ds(v * 16, 16)
        dv = degb[sl]
        bits = lax.bitcast_convert_type(dv, jnp.int32)
        y = lax.bitcast_convert_type(
            jnp.full((16,), 0x5F3759DF, jnp.int32) - (bits >> 1), jnp.float32)
        half = dv * 0.5
        for _i in range(4):
            y = y * (1.5 - half * y * y)
        degb[sl] = y
        return 0
    lax.fori_loop(0, STRIPE // 16, newton, 0)
    pltpu.sync_copy(degb, dis_sh.at[pl.ds(st, STRIPE)])

    @pl.when(c == 0)
    def _():
        pltpu.sync_copy(degb, dis_h.at[pl.ds(st, STRIPE)])
    plsc.subcore_barrier()

    pltpu.sync_copy(dis_sh, disf)

    def nrm(j, _):
        def inner(k, _2):
            sl = pl.ds(k * 16, 16)
            dr = plsc.load_gather(disf, [row2d[j, sl]])
            dc = plsc.load_gather(disf, [col2d[j, sl]])
            ew2d[j, sl] = dr * ew2d[j, sl] * dc
            return 0
        lax.fori_loop(0, BW // 16, inner, 0)
        return 0
    lax.fori_loop(0, NB, nrm, 0)
    pltpu.sync_copy(ew2d, norm_h.at[wid])

    _agg_loop(row2d, col2d, ew2d, h_sh, acc_sh,
              (gb0, gb1, gb2, gb3), (sb0, sb1, sb2, sb3),
              (gsem0, gsem1, gsem2, gsem3), (ssem0, ssem1, ssem2, ssem3))
    plsc.subcore_barrier()
    pltpu.sync_copy(acc_sh.at[pl.ds(st, STRIPE)],
                    p_h.at[c].at[pl.ds(st, STRIPE)])


@functools.partial(
    pl.kernel,
    out_type=jax.ShapeDtypeStruct((NC, NP, DH), jnp.float32),
    mesh=_MESH,
    compiler_params=pltpu.CompilerParams(needs_layout_passes=False, use_tc_tiling_on_sc=False),
    scratch_types=[
        pltpu.VMEM_SHARED((NP, DH), jnp.float32),
        pltpu.VMEM_SHARED((NP, DH), jnp.float32),
        pltpu.VMEM((NB, BW), jnp.int32),
        pltpu.VMEM((NB, BW), jnp.int32),
        pltpu.VMEM((NB, BW), jnp.float32),
        pltpu.VMEM((BW, DH), jnp.float32),
        pltpu.VMEM((BW, DH), jnp.float32),
        pltpu.VMEM((BW, DH), jnp.float32),
        pltpu.VMEM((BW, DH), jnp.float32),
        pltpu.VMEM((BW, DH), jnp.float32),
        pltpu.VMEM((BW, DH), jnp.float32),
        pltpu.VMEM((BW, DH), jnp.float32),
        pltpu.VMEM((BW, DH), jnp.float32),
        pltpu.SemaphoreType.DMA,
        pltpu.SemaphoreType.DMA,
        pltpu.SemaphoreType.DMA,
        pltpu.SemaphoreType.DMA,
        pltpu.SemaphoreType.DMA,
        pltpu.SemaphoreType.DMA,
        pltpu.SemaphoreType.DMA,
        pltpu.SemaphoreType.DMA,
        pltpu.SemaphoreType.DMA,
    ],
)
def _sc_layer2(row_h, col_h, norm_h, h1_h, q_h,
               acc_sh, h_sh, row2d, col2d, norm2d,
               gb0, gb1, gb2, gb3, sb0, sb1, sb2, sb3,
               gsem0, gsem1, gsem2, gsem3, ssem0, ssem1, ssem2, ssem3,
               hsem):
    c = lax.axis_index("c")
    s = lax.axis_index("s")
    wid = s * 2 + c
    st = s * STRIPE

    pltpu.sync_copy(row_h.at[wid], row2d)
    pltpu.sync_copy(col_h.at[wid], col2d)
    pltpu.sync_copy(norm_h.at[wid], norm2d)
    pltpu.sync_copy(h1_h.at[pl.ds(st, STRIPE)], h_sh.at[pl.ds(st, STRIPE)])
    _zero_rows(gb0, BW)
    for k in range(STRIPE // BW):
        pltpu.sync_copy(gb0, acc_sh.at[pl.ds(st + k * BW, BW)])
    plsc.subcore_barrier()

    _agg_loop(row2d, col2d, norm2d, h_sh, acc_sh,
              (gb0, gb1, gb2, gb3), (sb0, sb1, sb2, sb3),
              (gsem0, gsem1, gsem2, gsem3), (ssem0, ssem1, ssem2, ssem3))
    plsc.subcore_barrier()
    pltpu.sync_copy(acc_sh.at[pl.ds(st, STRIPE)],
                    q_h.at[c].at[pl.ds(st, STRIPE)])


def _mm_body(x_ref, w_ref, o_ref):
    o_ref[...] = jnp.dot(x_ref[...], w_ref[...],
                         preferred_element_type=jnp.float32)


_tc_matmul = pl.pallas_call(
    _mm_body,
    out_shape=jax.ShapeDtypeStruct((NP, DH), jnp.float32),
)


def _comb_body(p_ref, dis_ref, h0_ref, o_ref):
    d2 = dis_ref[...] * dis_ref[...]
    o_ref[...] = p_ref[0] + p_ref[1] + d2 * h0_ref[...]


_tc_combine = pl.pallas_call(
    _comb_body,
    out_shape=jax.ShapeDtypeStruct((NP, DH), jnp.float32),
)


def _final_body(q_ref, h1_ref, dis_ref, w2_ref, o_ref):
    d2 = dis_ref[...] * dis_ref[...]
    h2 = q_ref[0] + q_ref[1] + d2 * h1_ref[...]
    logits = jnp.dot(h2, w2_ref[...], preferred_element_type=jnp.float32)
    m = jnp.max(logits, axis=-1, keepdims=True)
    sh = logits - m
    lse = jnp.log(jnp.sum(jnp.exp(sh), axis=-1, keepdims=True))
    o_ref[...] = sh - lse


_tc_final = pl.pallas_call(
    _final_body,
    out_shape=jax.ShapeDtypeStruct((NP, NCLS), jnp.float32),
)


def kernel(x, edge_index, edge_weight, W1, W2):
    row = edge_index[0].astype(jnp.int32)
    col = edge_index[1].astype(jnp.int32)
    pad = EP - E
    rowp = jnp.concatenate([row, jnp.zeros((pad,), jnp.int32)]).reshape(NT, NB, BW)
    colp = jnp.concatenate([col, jnp.zeros((pad,), jnp.int32)]).reshape(NT, NB, BW)
    ewp = jnp.concatenate(
        [edge_weight.astype(jnp.float32), jnp.zeros((pad,), jnp.float32)]
    ).reshape(NT, NB, BW)
    xp = jnp.pad(x.astype(jnp.float32), ((0, NP - N), (0, 0)))

    h0 = _tc_matmul(xp, W1)
    p, dis, normv = _sc_layer1(rowp, colp, ewp, h0)
    dis2d = dis.reshape(NP, 1)
    h1 = _tc_combine(p, dis2d, h0)
    q = _sc_layer2(rowp, colp, normv, h1)
    out = _tc_final(q, h1, dis2d, W2)
    return out[:N]
